# Initial kernel scaffold; baseline (speedup 1.0000x reference)
import jax
import jax.numpy as jnp
from jax.experimental import pallas as pl


def kernel(x, phase_signatures):
    raise NotImplementedError("write your pallas kernel here")


# speedup vs baseline: 1.0531x; 1.0531x over previous
import functools

import jax
import jax.numpy as jnp
from jax.experimental import pallas as pl

_NEG = -3.0e38


def _router_block(x_ref, s_ref, w_ref, i_ref):
    xb = x_ref[...]
    s = s_ref[...]
    n2 = jnp.sum(xb * xb, axis=1, keepdims=True)
    inv = 1.0 / jnp.maximum(jnp.sqrt(n2), 1e-12)
    xn = xb * inv
    y = jax.lax.dot_general(
        xn, s, (((1,), (1,)), ((), ())),
        preferred_element_type=jnp.float32)

    idx = jax.lax.broadcasted_iota(jnp.int32, y.shape, 1)
    m1 = jnp.max(y, axis=1, keepdims=True)
    i1 = jnp.min(jnp.where(y == m1, idx, 127), axis=1, keepdims=True)
    y2 = jnp.where(idx == i1, _NEG, y)
    m2 = jnp.max(y2, axis=1, keepdims=True)
    i2 = jnp.min(jnp.where(y2 == m2, idx, 127), axis=1, keepdims=True)

    w = jnp.concatenate([m1, m2], axis=1)
    w_ref[...] = jnp.maximum(w, 0.0)
    i_ref[...] = jnp.concatenate([i1, i2], axis=1)


@functools.partial(jax.jit, static_argnames=())
def kernel(x, phase_signatures):
    n, d = x.shape
    e = phase_signatures.shape[0]
    block = 4096
    grid = (n // block,)
    w, i = pl.pallas_call(
        _router_block,
        grid=grid,
        in_specs=[
            pl.BlockSpec((block, d), lambda b: (b, 0)),
            pl.BlockSpec((e, d), lambda b: (0, 0)),
        ],
        out_specs=[
            pl.BlockSpec((block, 2), lambda b: (b, 0)),
            pl.BlockSpec((block, 2), lambda b: (b, 0)),
        ],
        out_shape=[
            jax.ShapeDtypeStruct((n, 2), jnp.float32),
            jax.ShapeDtypeStruct((n, 2), jnp.int32),
        ],
    )(x, phase_signatures)
    return (w, i)


# speedup vs baseline: 2.1135x; 2.0070x over previous
import functools

import jax
import jax.numpy as jnp
from jax.experimental import pallas as pl

_IMIN = -2147483648


def _router_block(x_ref, s_ref, w_ref, i_ref):
    xb = x_ref[...]
    s = s_ref[...]
    n2 = jnp.sum(xb * xb, axis=1, keepdims=True)
    inv = jax.lax.rsqrt(jnp.maximum(n2, 1e-24))
    xn = xb * inv
    y = jax.lax.dot_general(
        s, xn, (((1,), (1,)), ((), ())),
        preferred_element_type=jnp.float32)

    bits = jax.lax.bitcast_convert_type(y, jnp.int32)
    key = bits ^ ((bits >> 31) & 0x7FFFFFFF)
    rank = 7 - jax.lax.broadcasted_iota(jnp.int32, y.shape, 0)
    packed = (key & ~7) | rank

    p1 = jnp.max(packed, axis=0, keepdims=True)
    p2 = jnp.max(jnp.where(packed == p1, _IMIN, packed), axis=0, keepdims=True)

    pv = jnp.concatenate([p1, p2], axis=0)
    i_ref[...] = 7 - (pv & 7)
    vbits = pv & ~7
    w = jax.lax.bitcast_convert_type(
        vbits ^ ((vbits >> 31) & 0x7FFFFFFF), jnp.float32)
    w_ref[...] = jnp.maximum(w, 0.0)


@functools.partial(jax.jit, static_argnames=())
def kernel(x, phase_signatures):
    n, d = x.shape
    e = phase_signatures.shape[0]
    block = 4096
    grid = (n // block,)
    w_t, i_t = pl.pallas_call(
        _router_block,
        grid=grid,
        in_specs=[
            pl.BlockSpec((block, d), lambda b: (b, 0)),
            pl.BlockSpec((e, d), lambda b: (0, 0)),
        ],
        out_specs=[
            pl.BlockSpec((2, block), lambda b: (0, b)),
            pl.BlockSpec((2, block), lambda b: (0, b)),
        ],
        out_shape=[
            jax.ShapeDtypeStruct((2, n), jnp.float32),
            jax.ShapeDtypeStruct((2, n), jnp.int32),
        ],
    )(x, phase_signatures)
    return (w_t.T, i_t.T)
